# speedup vs baseline: 1.6638x; 1.6638x over previous
import functools

import jax
import jax.numpy as jnp
from jax import lax
from jax.experimental import pallas as pl
from jax.experimental.pallas import tpu as pltpu
from jax.experimental.pallas import tpu_sc as plsc

VOCAB = 512
HIDDEN = 2048
B_TOTAL = 4 * 8192

NUM_CORES = 2
NUM_SUBCORES = 16
NUM_WORKERS = NUM_CORES * NUM_SUBCORES
B_PER_W = B_TOTAL // NUM_WORKERS
K = 16
NBUF = 2
NCHUNK = B_PER_W // K

_mesh = plsc.VectorSubcoreMesh(core_axis_name="c", subcore_axis_name="s")


@functools.partial(
    pl.kernel,
    mesh=_mesh,
    out_type=jax.ShapeDtypeStruct((B_TOTAL, HIDDEN), jnp.float32),
    scratch_types=[
        pltpu.VMEM((B_PER_W,), jnp.int32),
        pltpu.VMEM((NBUF, K, HIDDEN), jnp.float32),
        pltpu.SemaphoreType.DMA,
        pltpu.SemaphoreType.DMA,
    ],
)
def _gather_kernel(table_hbm, idx_hbm, out_hbm, idx_v, rows_v, gsem, wsem):
    wid = lax.axis_index("s") * NUM_CORES + lax.axis_index("c")
    base = wid * B_PER_W

    pltpu.sync_copy(idx_hbm.at[pl.ds(base, B_PER_W)], idx_v)

    def start_gather(c, b):
        pltpu.async_copy(
            table_hbm.at[idx_v.at[pl.ds(c * K, K)]], rows_v.at[b], gsem
        )

    def wait_gather(b):
        pltpu.make_async_copy(
            table_hbm.at[idx_v.at[pl.ds(0, K)]], rows_v.at[b], gsem
        ).wait()

    def start_write(c, b):
        pltpu.async_copy(
            rows_v.at[b], out_hbm.at[pl.ds(base + c * K, K)], wsem
        )

    def wait_write(b):
        pltpu.make_async_copy(
            rows_v.at[b], out_hbm.at[pl.ds(base, K)], wsem
        ).wait()

    for b in range(NBUF):
        start_gather(b, b)

    def outer(i, _):
        c0 = i * NBUF
        for b in range(NBUF):
            c = c0 + b
            wait_gather(b)
            start_write(c, b)
            nxt = c + NBUF

            @pl.when(nxt < NCHUNK)
            def _():
                wait_write(b)
                start_gather(nxt, b)

        return 0

    lax.fori_loop(0, NCHUNK // NBUF, outer, 0)

    for b in range(NBUF):
        wait_write(b)


def kernel(input_ids, table):
    flat_ids = input_ids.reshape(B_TOTAL).astype(jnp.int32)
    out = _gather_kernel(table, flat_ids)
    return out.reshape(input_ids.shape[0], input_ids.shape[1], HIDDEN)
